# baseline (device time: 38101 ns/iter reference)
import jax
import jax.numpy as jnp
from jax import lax
from jax.experimental import pallas as pl
from jax.experimental.pallas import tpu as pltpu

N_DEV = 4
DH = 64


def kernel(x, Wq, Wo, Wk, Wv):
    B, Sq, D = x.shape
    d_sh = Wq.shape[1]
    H_sh = d_sh // DH
    R = B * Sq

    def body(x_ref, wq_ref, wo_ref, wk_ref, wv_ref, out_ref,
             acc_ref, comm_ref, send_sems, recv_sems):
        my = lax.axis_index("i")
        left = lax.rem(my + N_DEV - 1, N_DEV)
        right = lax.rem(my + 1, N_DEV)

        barrier_sem = pltpu.get_barrier_semaphore()
        for nbr in (left, right):
            pl.semaphore_signal(
                barrier_sem, inc=1,
                device_id=(nbr,), device_id_type=pl.DeviceIdType.MESH,
            )
        pl.semaphore_wait(barrier_sem, 2)

        xf = x_ref[...].reshape(R, D)
        q = jnp.dot(xf, wq_ref[...], preferred_element_type=jnp.float32)
        k = jnp.dot(xf, wk_ref[...], preferred_element_type=jnp.float32)
        v = jnp.dot(xf, wv_ref[...], preferred_element_type=jnp.float32)

        for b in range(B):
            rows = slice(b * Sq, (b + 1) * Sq)
            for h in range(H_sh):
                cols = slice(h * DH, (h + 1) * DH)
                qh = q[rows, cols]
                kh = k[rows, cols]
                vh = v[rows, cols]
                s = jnp.dot(qh, kh.T, preferred_element_type=jnp.float32) * 0.125
                m = jnp.max(s, axis=-1, keepdims=True)
                p = jnp.exp(s - m)
                ssum = jnp.sum(p, axis=-1, keepdims=True)
                acc_ref[rows, cols] = jnp.dot(
                    p / ssum, vh, preferred_element_type=jnp.float32
                )

        partial = jnp.dot(
            acc_ref[...], wo_ref[...], preferred_element_type=jnp.float32
        )
        comm_ref[0] = partial

        total = partial
        for hop in range(N_DEV - 1):
            rdma = pltpu.make_async_remote_copy(
                src_ref=comm_ref.at[hop],
                dst_ref=comm_ref.at[hop + 1],
                send_sem=send_sems.at[hop],
                recv_sem=recv_sems.at[hop],
                device_id=(right,),
                device_id_type=pl.DeviceIdType.MESH,
            )
            rdma.start()
            rdma.wait()
            total = total + comm_ref[hop + 1]

        out_ref[...] = total.reshape(B, Sq, D)

    return pl.pallas_call(
        body,
        out_shape=jax.ShapeDtypeStruct((B, Sq, D), jnp.float32),
        in_specs=[pl.BlockSpec(memory_space=pltpu.VMEM)] * 5,
        out_specs=pl.BlockSpec(memory_space=pltpu.VMEM),
        scratch_shapes=[
            pltpu.VMEM((R, d_sh), jnp.float32),
            pltpu.VMEM((N_DEV, R, D), jnp.float32),
            pltpu.SemaphoreType.DMA((N_DEV - 1,)),
            pltpu.SemaphoreType.DMA((N_DEV - 1,)),
        ],
        compiler_params=pltpu.CompilerParams(collective_id=0),
    )(x, Wq, Wo, Wk, Wv)


# device time: 13111 ns/iter; 2.9060x vs baseline; 2.9060x over previous
import jax
import jax.numpy as jnp
from jax import lax
from jax.experimental import pallas as pl
from jax.experimental.pallas import tpu as pltpu

N_DEV = 4
DH = 64


def kernel(x, Wq, Wo, Wk, Wv):
    B, Sq, D = x.shape
    d_sh = Wq.shape[1]
    H_sh = d_sh // DH
    R = B * Sq

    def body(x_ref, wq_ref, wo_ref, wk_ref, wv_ref, out_ref,
             acc_ref, comm_ref, send_sems, recv_sems):
        my = lax.axis_index("i")
        left = lax.rem(my + N_DEV - 1, N_DEV)
        right = lax.rem(my + 1, N_DEV)

        barrier_sem = pltpu.get_barrier_semaphore()
        for nbr in (left, right):
            pl.semaphore_signal(
                barrier_sem, inc=1,
                device_id=(nbr,), device_id_type=pl.DeviceIdType.MESH,
            )
        pl.semaphore_wait(barrier_sem, 2)

        xf = x_ref[...].reshape(R, D)
        q = jnp.dot(xf, wq_ref[...], preferred_element_type=jnp.float32)
        k = jnp.dot(xf, wk_ref[...], preferred_element_type=jnp.float32)
        v = jnp.dot(xf, wv_ref[...], preferred_element_type=jnp.float32)

        for b in range(B):
            rows = slice(b * Sq, (b + 1) * Sq)
            for h in range(H_sh):
                cols = slice(h * DH, (h + 1) * DH)
                qh = q[rows, cols]
                kh = k[rows, cols]
                vh = v[rows, cols]
                s = jnp.dot(qh, kh.T, preferred_element_type=jnp.float32) * 0.125
                m = jnp.max(s, axis=-1, keepdims=True)
                p = jnp.exp(s - m)
                ssum = jnp.sum(p, axis=-1, keepdims=True)
                acc_ref[rows, cols] = jnp.dot(
                    p / ssum, vh, preferred_element_type=jnp.float32
                )

        partial = jnp.dot(
            acc_ref[...], wo_ref[...], preferred_element_type=jnp.float32
        )
        comm_ref[0] = partial

        total = partial
        for hop in range(N_DEV - 1):
            rdma = pltpu.make_async_remote_copy(
                src_ref=comm_ref.at[hop],
                dst_ref=comm_ref.at[hop + 1],
                send_sem=send_sems.at[hop],
                recv_sem=recv_sems.at[hop],
                device_id=(right,),
                device_id_type=pl.DeviceIdType.MESH,
            )
            rdma.start()
            rdma.wait()
            total = total + comm_ref[hop + 1]

        out_ref[...] = total.reshape(B, Sq, D)

    return pl.pallas_call(
        body,
        out_shape=jax.ShapeDtypeStruct((B, Sq, D), jnp.float32),
        in_specs=[pl.BlockSpec(memory_space=pltpu.VMEM)] * 5,
        out_specs=pl.BlockSpec(memory_space=pltpu.VMEM),
        scratch_shapes=[
            pltpu.VMEM((R, d_sh), jnp.float32),
            pltpu.VMEM((N_DEV, R, D), jnp.float32),
            pltpu.SemaphoreType.DMA((N_DEV - 1,)),
            pltpu.SemaphoreType.DMA((N_DEV - 1,)),
        ],
    )(x, Wq, Wo, Wk, Wv)
